# baseline (speedup 1.0000x reference)
import functools

import jax
import jax.numpy as jnp
from jax import lax
from jax.experimental import pallas as pl
from jax.experimental.pallas import tpu as pltpu
from jax.experimental.pallas import tpu_sc as plsc

N = 10000
NPAD = 10240
E = 320000
G = 128
TEMP = 5.0
CH = 100
NCHUNK = E // CH
NB = 512
NGRID = NPAD // NB
EB = 2000
ZR = NPAD // 16

_f32 = jnp.float32
_SDS = jax.ShapeDtypeStruct


def _mesh():
    return plsc.VectorSubcoreMesh(core_axis_name="c", subcore_axis_name="s")


def _mm_body(x_ref, w_ref, o_ref):
    o_ref[...] = jnp.dot(x_ref[...], w_ref[...], preferred_element_type=_f32)


def tc_matmul(x, w):
    n, din = x.shape
    dout = w.shape[1]
    return pl.pallas_call(
        _mm_body,
        grid=(n // NB,),
        in_specs=[pl.BlockSpec((NB, din), lambda i: (i, 0)),
                  pl.BlockSpec((din, dout), lambda i: (0, 0))],
        out_specs=pl.BlockSpec((NB, dout), lambda i: (i, 0)),
        out_shape=_SDS((n, dout), _f32),
    )(x, w)


def _gin_layer_body(last, hw_ref, p0_ref, p1_ref, w2_ref, b1_ref, b2_ref,
                    eps_ref, w1n_ref, *out_refs):
    u = hw_ref[...] * eps_ref[0, 0] + p0_ref[...] + p1_ref[...] + b1_ref[...]
    v = jnp.maximum(u, 0.0)
    h = jnp.maximum(jnp.dot(v, w2_ref[...], preferred_element_type=_f32)
                    + b2_ref[...], 0.0)
    if last:
        out_refs[0][...] = h
    else:
        out_refs[0][...] = h
        out_refs[1][...] = jnp.dot(h, w1n_ref[...], preferred_element_type=_f32)


def tc_gin_layer(hW, p0, p1, p, w1_next):
    last = w1_next is None
    wspec = lambda shp: pl.BlockSpec(shp, lambda i: (0, 0))
    in_specs = [pl.BlockSpec((NB, 64), lambda i: (i, 0))] * 3 + [
        wspec((64, 64)), wspec((1, 64)), wspec((1, 64)),
        pl.BlockSpec(memory_space=pltpu.SMEM), wspec((64, 64))]
    out_specs = [pl.BlockSpec((NB, 64), lambda i: (i, 0))] * (1 if last else 2)
    out_shape = [_SDS((NPAD, 64), _f32)] * (1 if last else 2)
    epsp = (1.0 + p["eps"]).reshape(1, 1).astype(_f32)
    w1n = jnp.zeros((64, 64), _f32) if last else w1_next
    outs = pl.pallas_call(
        functools.partial(_gin_layer_body, last),
        grid=(NGRID,),
        in_specs=in_specs,
        out_specs=out_specs,
        out_shape=out_shape,
    )(hW, p0, p1, p["W2"], p["b1"].reshape(1, 64), p["b2"].reshape(1, 64),
      epsp, w1n)
    return outs[0] if last else outs


def _masks_body(x_ref, z_ref, nw1_ref, nb1_ref, nw2_ref, nb2_ref,
                fw1_ref, fb1_ref, fw2_0, fw2_1, fw2_2, fw2_3,
                fb2_0, fb2_1, fb2_2, fb2_3, w1c_ref,
                nm_ref, fm_ref, t0_ref, t1_ref, t2_ref, t3_ref):
    z = z_ref[...]
    x = x_ref[...]
    h1 = jnp.maximum(jnp.dot(z, nw1_ref[...], preferred_element_type=_f32)
                     + nb1_ref[...], 0.0)
    t = h1 * nw2_ref[...]
    nm_parts = [jnp.sum(t[:, k * 64:(k + 1) * 64], axis=1, keepdims=True)
                for k in range(4)]
    nm = jax.nn.sigmoid((jnp.concatenate(nm_parts, axis=1) + nb2_ref[...])
                        / TEMP)
    nm_ref[...] = nm
    h2 = jnp.maximum(jnp.dot(z, fw1_ref[...], preferred_element_type=_f32)
                     + fb1_ref[...], 0.0)
    fw2 = (fw2_0, fw2_1, fw2_2, fw2_3)
    fb2 = (fb2_0, fb2_1, fb2_2, fb2_3)
    fms = []
    m1s = []
    for k in range(4):
        fmk = jax.nn.sigmoid(
            (jnp.dot(h2[:, k * 64:(k + 1) * 64], fw2[k][...],
                     preferred_element_type=_f32) + fb2[k][...]) / TEMP)
        fms.append(fmk)
        mx = x * nm[:, k:k + 1] * fmk
        m1s.append(jnp.dot(mx, w1c_ref[...], preferred_element_type=_f32))
    fm_ref[...] = jnp.concatenate(fms, axis=1)
    t0_ref[...] = m1s[0]
    t1_ref[...] = m1s[1]
    t2_ref[...] = m1s[2]
    t3_ref[...] = m1s[3]


def tc_masks(x_pad, Z, params):
    nm_p = params["node_masks"]
    fm_p = params["feat_masks"]
    w1c = params["classifier"][0]["W1"]
    nw1 = jnp.concatenate([p["W1"] for p in nm_p], axis=1)
    nb1 = jnp.concatenate([p["b1"] for p in nm_p]).reshape(1, 256)
    nw2 = jnp.concatenate([p["W2"][:, 0] for p in nm_p]).reshape(1, 256)
    nb2 = jnp.stack([p["b2"][0] for p in nm_p]).reshape(1, 4)
    fw1 = jnp.concatenate([p["W1"] for p in fm_p], axis=1)
    fb1 = jnp.concatenate([p["b1"] for p in fm_p]).reshape(1, 256)
    wspec = lambda shp: pl.BlockSpec(shp, lambda i: (0, 0))
    in_specs = [pl.BlockSpec((NB, 128), lambda i: (i, 0)),
                pl.BlockSpec((NB, 64), lambda i: (i, 0)),
                wspec((64, 256)), wspec((1, 256)), wspec((1, 256)),
                wspec((1, 4)), wspec((64, 256)), wspec((1, 256)),
                wspec((64, 128)), wspec((64, 128)), wspec((64, 128)),
                wspec((64, 128)), wspec((1, 128)), wspec((1, 128)),
                wspec((1, 128)), wspec((1, 128)), wspec((128, 64))]
    out_specs = [pl.BlockSpec((NB, 4), lambda i: (i, 0)),
                 pl.BlockSpec((NB, 512), lambda i: (i, 0))] + [
                 pl.BlockSpec((NB, 64), lambda i: (i, 0))] * 4
    out_shape = [_SDS((NPAD, 4), _f32), _SDS((NPAD, 512), _f32)] + [
                 _SDS((NPAD, 64), _f32)] * 4
    args = [x_pad, Z, nw1, nb1, nw2, nb2, fw1, fb1]
    args += [fm_p[k]["W2"] for k in range(4)]
    args += [fm_p[k]["b2"].reshape(1, 128) for k in range(4)]
    args += [w1c]
    return pl.pallas_call(
        _masks_body, grid=(NGRID,), in_specs=in_specs, out_specs=out_specs,
        out_shape=out_shape)(*args)


def _em_body(efs_ref, efd_ref, wa_ref, wb_ref, b1_ref, w2_ref, b2_ref, em_ref):
    h = jnp.maximum(
        jnp.dot(efs_ref[...], wa_ref[...], preferred_element_type=_f32)
        + jnp.dot(efd_ref[...], wb_ref[...], preferred_element_type=_f32)
        + b1_ref[...], 0.0)
    t = h * w2_ref[...]
    parts = [jnp.sum(t[:, k * 64:(k + 1) * 64], axis=1, keepdims=True)
             for k in range(4)]
    em_ref[...] = jax.nn.sigmoid(
        (jnp.concatenate(parts, axis=1) + b2_ref[...]) / TEMP)


def tc_em(efs, efd, params):
    em_p = params["edge_masks"]
    wa = jnp.concatenate([p["W1"][:64] for p in em_p], axis=1)
    wb = jnp.concatenate([p["W1"][64:] for p in em_p], axis=1)
    b1 = jnp.concatenate([p["b1"] for p in em_p]).reshape(1, 256)
    w2 = jnp.concatenate([p["W2"][:, 0] for p in em_p]).reshape(1, 256)
    b2 = jnp.stack([p["b2"][0] for p in em_p]).reshape(1, 4)
    wspec = lambda shp: pl.BlockSpec(shp, lambda i: (0, 0))
    return pl.pallas_call(
        _em_body,
        grid=(E // EB,),
        in_specs=[pl.BlockSpec((EB, 64), lambda i: (i, 0)),
                  pl.BlockSpec((EB, 64), lambda i: (i, 0)),
                  wspec((64, 256)), wspec((64, 256)), wspec((1, 256)),
                  wspec((1, 256)), wspec((1, 4))],
        out_specs=pl.BlockSpec((EB, 4), lambda i: (i, 0)),
        out_shape=_SDS((E, 4), _f32),
    )(efs, efd, wa, wb, b1, w2, b2)


def _cls_layer_body(last, t0_ref, t1_ref, t2_ref, t3_ref,
                    u0_ref, u1_ref, u2_ref, u3_ref, w2_ref, b1_ref,
                    b2_ref, eps_ref, w1n_ref, *out_refs):
    w2 = w2_ref[...]
    b2 = b2_ref[...]
    t_refs = (t0_ref, t1_ref, t2_ref, t3_ref)
    u_refs = (u0_ref, u1_ref, u2_ref, u3_ref)
    hs = []
    for k in range(4):
        u = t_refs[k][...] * eps_ref[0, 0] + u_refs[k][...] + b1_ref[...]
        v = jnp.maximum(u, 0.0)
        hs.append(jnp.maximum(jnp.dot(v, w2, preferred_element_type=_f32)
                              + b2, 0.0))
    if last:
        out_refs[0][...] = jnp.concatenate(hs, axis=1)
    else:
        w1n = w1n_ref[...]
        for k in range(4):
            out_refs[k][...] = jnp.dot(hs[k], w1n,
                                       preferred_element_type=_f32)


def tc_cls_layer(Ts, Us, p, w1_next):
    last = w1_next is None
    wspec = lambda shp: pl.BlockSpec(shp, lambda i: (0, 0))
    in_specs = [pl.BlockSpec((NB, 64), lambda i: (i, 0))] * 8 + [
        wspec((64, 64)), wspec((1, 64)), wspec((1, 64)),
        pl.BlockSpec(memory_space=pltpu.SMEM), wspec((64, 64))]
    if last:
        out_specs = [pl.BlockSpec((NB, 256), lambda i: (i, 0))]
        out_shape = [_SDS((NPAD, 256), _f32)]
    else:
        out_specs = [pl.BlockSpec((NB, 64), lambda i: (i, 0))] * 4
        out_shape = [_SDS((NPAD, 64), _f32)] * 4
    epsp = (1.0 + p["eps"]).reshape(1, 1).astype(_f32)
    w1n = jnp.zeros((64, 64), _f32) if last else w1_next
    outs = pl.pallas_call(
        functools.partial(_cls_layer_body, last),
        grid=(NGRID,),
        in_specs=in_specs,
        out_specs=out_specs,
        out_shape=out_shape,
    )(*Ts, *Us, p["W2"], p["b1"].reshape(1, 64), p["b2"].reshape(1, 64),
      epsp, w1n)
    return outs[0] if last else outs


def _pool_body(z_ref, mz_ref, b_ref, wl_ref, bl_ref,
               ho_ref, hs_ref, lg_ref, sz_acc, sm_acc, cnt_acc):
    i = pl.program_id(0)

    @pl.when(i == 0)
    def _init():
        sz_acc[...] = jnp.zeros_like(sz_acc)
        sm_acc[...] = jnp.zeros_like(sm_acc)
        cnt_acc[...] = jnp.zeros_like(cnt_acc)

    bcol = jnp.reshape(b_ref[...], (NB, 1))
    iota = lax.broadcasted_iota(jnp.int32, (NB, G), 1)
    rows = i * NB + lax.broadcasted_iota(jnp.int32, (NB, 1), 0)
    valid = (rows < N).astype(_f32)
    oh = (bcol == iota).astype(_f32) * valid
    dn = (((0,), (0,)), ((), ()))
    sz_acc[...] += lax.dot_general(oh, z_ref[...], dn,
                                   preferred_element_type=_f32)
    sm_acc[...] += lax.dot_general(oh, mz_ref[...], dn,
                                   preferred_element_type=_f32)
    cnt_acc[...] += lax.dot_general(oh, valid, dn,
                                    preferred_element_type=_f32)

    @pl.when(i == NGRID - 1)
    def _fin():
        cnt = jnp.maximum(cnt_acc[...], 1.0)
        ho = sz_acc[...] / cnt
        hs = sm_acc[...] / cnt
        ho_ref[...] = ho
        hs_ref[...] = hs
        wl = wl_ref[...]
        bl = bl_ref[...]
        lgs = [jnp.dot(hs[:, k * 64:(k + 1) * 64], wl[:, k * 10:(k + 1) * 10],
                       preferred_element_type=_f32) + bl[:, k * 10:(k + 1) * 10]
               for k in range(4)]
        lg_ref[...] = jnp.concatenate(lgs, axis=1)


def tc_pool(Z, mZ, batch3, params):
    wl = jnp.concatenate([params["classifiers"][k]["W"] for k in range(4)],
                         axis=1)
    bl = jnp.concatenate([params["classifiers"][k]["b"] for k in range(4)]
                         ).reshape(1, 40)
    wspec = lambda shp: pl.BlockSpec(shp, lambda i: (0, 0))
    return pl.pallas_call(
        _pool_body,
        grid=(NGRID,),
        in_specs=[pl.BlockSpec((NB, 64), lambda i: (i, 0)),
                  pl.BlockSpec((NB, 256), lambda i: (i, 0)),
                  pl.BlockSpec((1, NB, 1), lambda i: (i, 0, 0)),
                  wspec((64, 40)), wspec((1, 40))],
        out_specs=[pl.BlockSpec((G, 64), lambda i: (0, 0)),
                   pl.BlockSpec((G, 256), lambda i: (0, 0)),
                   pl.BlockSpec((G, 40), lambda i: (0, 0))],
        out_shape=[_SDS((G, 64), _f32), _SDS((G, 256), _f32),
                   _SDS((G, 40), _f32)],
        scratch_shapes=[pltpu.VMEM((G, 64), _f32), pltpu.VMEM((G, 256), _f32),
                        pltpu.VMEM((G, 1), _f32)],
    )(Z, mZ, batch3, wl, bl)


_GDN = lax.GatherDimensionNumbers(offset_dims=(), collapsed_slice_dims=(0,),
                                  start_index_map=(0,))


def _lane_bcast(w, idx):
    return lax.gather(w, idx[:, None], _GDN, (1,),
                      mode=lax.GatherScatterMode.PROMISE_IN_BOUNDS)

S = 2


def _make_sc_causal():
    cpw = NCHUNK // 32
    nsup = cpw // S

    def body(tbl, srcr, dstr, zr, out0, out1,
             src_v, dst_v, rows0, rows1, accum, sem0, sem1, ssem0, ssem1):
        c = lax.axis_index("c")
        s = lax.axis_index("s")
        wid = s * 2 + c
        pltpu.sync_copy(zr.at[pl.ds(s * ZR, ZR)], accum.at[pl.ds(s * ZR, ZR)])
        pltpu.sync_copy(srcr.at[wid], src_v)
        pltpu.sync_copy(dstr.at[wid], dst_v)
        plsc.subcore_barrier()

        def g_start(J, buf, sem):
            for k in range(S):
                pltpu.async_copy(tbl.at[src_v.at[J * S + k]], buf.at[k], sem)

        def g_wait(buf, sem):
            for k in range(S):
                pltpu.make_async_copy(tbl.at[src_v.at[0]], buf.at[k],
                                      sem).wait()

        def s_start(J, buf, ssem):
            for k in range(S):
                pltpu.async_copy(buf.at[k], accum.at[dst_v.at[J * S + k]],
                                 ssem, add=True)

        def s_wait(buf, ssem):
            for k in range(S):
                pltpu.make_async_copy(buf.at[k], accum.at[dst_v.at[0]],
                                      ssem).wait()

        g_start(0, rows0, sem0)
        g_start(1, rows1, sem1)

        def pair(i, carry):
            J0 = i * 2
            g_wait(rows0, sem0)
            s_start(J0, rows0, ssem0)
            s_wait(rows0, ssem0)

            @pl.when(J0 + 2 < nsup)
            def _():
                g_start(J0 + 2, rows0, sem0)

            g_wait(rows1, sem1)
            s_start(J0 + 1, rows1, ssem1)
            s_wait(rows1, ssem1)

            @pl.when(J0 + 3 < nsup)
            def _():
                g_start(J0 + 3, rows1, sem1)
            return carry

        lax.fori_loop(0, nsup // 2, pair, 0)
        plsc.subcore_barrier()
        sl = pl.ds(s * ZR, ZR)

        @pl.when(c == 0)
        def _():
            pltpu.sync_copy(accum.at[sl], out0.at[sl])

        @pl.when(c == 1)
        def _():
            pltpu.sync_copy(accum.at[sl], out1.at[sl])

    return pl.kernel(
        body,
        out_type=(_SDS((NPAD, 64), _f32), _SDS((NPAD, 64), _f32)),
        mesh=_mesh(),
        compiler_params=pltpu.CompilerParams(use_tc_tiling_on_sc=False),
        scratch_types=[pltpu.VMEM((NCHUNK // 32, CH), jnp.int32),
                       pltpu.VMEM((NCHUNK // 32, CH), jnp.int32),
                       pltpu.VMEM((S, CH, 64), _f32),
                       pltpu.VMEM((S, CH, 64), _f32),
                       pltpu.VMEM_SHARED((NPAD, 64), _f32),
                       pltpu.SemaphoreType.DMA, pltpu.SemaphoreType.DMA,
                       pltpu.SemaphoreType.DMA, pltpu.SemaphoreType.DMA],
    )


def _make_sc_ef():
    cpt = NCHUNK // 16
    nsup = cpt // S

    def body(tbl, idxr, out, idx_v, b0, b1, sem0, sem1, wsem0, wsem1):
        c = lax.axis_index("c")
        s = lax.axis_index("s")
        pltpu.sync_copy(idxr.at[c, s], idx_v)
        outc = out.at[c]

        def g_start(J, buf, sem):
            for k in range(S):
                pltpu.async_copy(tbl.at[idx_v.at[J * S + k]], buf.at[k], sem)

        def g_wait(buf, sem):
            for k in range(S):
                pltpu.make_async_copy(tbl.at[idx_v.at[0]], buf.at[k],
                                      sem).wait()

        def w_start(J, buf, wsem):
            for k in range(S):
                pltpu.async_copy(buf.at[k], outc.at[s * cpt + J * S + k],
                                 wsem)

        def w_wait(buf, wsem):
            for k in range(S):
                pltpu.make_async_copy(buf.at[k], outc.at[0], wsem).wait()

        g_start(0, b0, sem0)
        g_start(1, b1, sem1)

        def pair(i, carry):
            J0 = i * 2
            g_wait(b0, sem0)
            w_start(J0, b0, wsem0)
            g_wait(b1, sem1)
            w_start(J0 + 1, b1, wsem1)

            @pl.when(J0 + 2 < nsup)
            def _():
                w_wait(b0, wsem0)
                g_start(J0 + 2, b0, sem0)

            @pl.when(J0 + 3 < nsup)
            def _():
                w_wait(b1, wsem1)
                g_start(J0 + 3, b1, sem1)
            return carry

        lax.fori_loop(0, nsup // 2, pair, 0)
        w_wait(b0, wsem0)
        w_wait(b1, wsem1)

    return pl.kernel(
        body,
        out_type=_SDS((2, NCHUNK, CH, 64), _f32),
        mesh=_mesh(),
        compiler_params=pltpu.CompilerParams(use_tc_tiling_on_sc=False),
        scratch_types=[pltpu.VMEM((NCHUNK // 16, CH), jnp.int32),
                       pltpu.VMEM((S, CH, 64), _f32),
                       pltpu.VMEM((S, CH, 64), _f32),
                       pltpu.SemaphoreType.DMA, pltpu.SemaphoreType.DMA,
                       pltpu.SemaphoreType.DMA, pltpu.SemaphoreType.DMA],
    )


def _make_sc_cls():
    cpt = NCHUNK // 16
    wrows = CH // 4
    nsup = cpt // S

    def body(tab, srcr, dstr, ewr, zr, uout,
             src_v, dst_v, rows0, rows1, ew0, ew1, accum,
             sem0, sem1, semw0, semw1, ssem0, ssem1):
        c = lax.axis_index("c")
        s = lax.axis_index("s")
        pltpu.sync_copy(srcr.at[s], src_v)
        pltpu.sync_copy(dstr.at[s], dst_v)

        def s_start(J, buf, ssem):
            for k in range(S):
                pltpu.async_copy(buf.at[k], accum.at[dst_v.at[J * S + k]],
                                 ssem, add=True)

        def s_wait(buf, ssem):
            for k in range(S):
                pltpu.make_async_copy(buf.at[k], accum.at[dst_v.at[0]],
                                      ssem).wait()

        def scale_k(buf, ewb, kidx):
            for kk in range(S):
                def grp(g, carry):
                    w = ewb[kk, g]
                    for i in range(4):
                        wk = _lane_bcast(w, jnp.full((16,), 4 * i + kidx,
                                                     jnp.int32))
                        r = g * 4 + i
                        for q in range(4):
                            sl = pl.ds(q * 16, 16)
                            buf[kk, r, sl] = buf[kk, r, sl] * wk
                    return carry

                lax.fori_loop(0, wrows, grp, 0, unroll=5)

        for p in range(2):
            tc_ = tab.at[2 * p + c]

            def g_start(J, buf, ewb, sem, semw, tc_=tc_):
                for k in range(S):
                    pltpu.async_copy(tc_.at[src_v.at[J * S + k]], buf.at[k],
                                     sem)
                pltpu.async_copy(ewr.at[pl.ds(s * cpt + J * S, S)], ewb, semw)

            def g_wait(buf, ewb, sem, semw, tc_=tc_):
                for k in range(S):
                    pltpu.make_async_copy(tc_.at[src_v.at[0]], buf.at[k],
                                          sem).wait()
                pltpu.make_async_copy(ewr.at[pl.ds(0, S)], ewb, semw).wait()

            def scale(buf, ewb, p=p):
                @pl.when(c == 0)
                def _():
                    scale_k(buf, ewb, 2 * p)

                @pl.when(c == 1)
                def _():
                    scale_k(buf, ewb, 2 * p + 1)

            plsc.subcore_barrier()
            pltpu.sync_copy(zr.at[pl.ds(s * ZR, ZR)],
                            accum.at[pl.ds(s * ZR, ZR)])
            plsc.subcore_barrier()

            g_start(0, rows0, ew0, sem0, semw0)
            g_start(1, rows1, ew1, sem1, semw1)

            def pair(i, carry, g_start=g_start, g_wait=g_wait, scale=scale):
                J0 = i * 2
                g_wait(rows0, ew0, sem0, semw0)
                scale(rows0, ew0)
                s_start(J0, rows0, ssem0)
                s_wait(rows0, ssem0)

                @pl.when(J0 + 2 < nsup)
                def _():
                    g_start(J0 + 2, rows0, ew0, sem0, semw0)

                g_wait(rows1, ew1, sem1, semw1)
                scale(rows1, ew1)
                s_start(J0 + 1, rows1, ssem1)
                s_wait(rows1, ssem1)

                @pl.when(J0 + 3 < nsup)
                def _():
                    g_start(J0 + 3, rows1, ew1, sem1, semw1)
                return carry

            lax.fori_loop(0, nsup // 2, pair, 0)
            plsc.subcore_barrier()
            sl = pl.ds(s * ZR, ZR)
            pltpu.sync_copy(accum.at[sl], uout.at[2 * p + c, sl])

    return pl.kernel(
        body,
        out_type=_SDS((4, NPAD, 64), _f32),
        mesh=_mesh(),
        compiler_params=pltpu.CompilerParams(use_tc_tiling_on_sc=False),
        scratch_types=[pltpu.VMEM((NCHUNK // 16, CH), jnp.int32),
                       pltpu.VMEM((NCHUNK // 16, CH), jnp.int32),
                       pltpu.VMEM((S, CH, 64), _f32),
                       pltpu.VMEM((S, CH, 64), _f32),
                       pltpu.VMEM((S, CH // 4, 16), _f32),
                       pltpu.VMEM((S, CH // 4, 16), _f32),
                       pltpu.VMEM_SHARED((NPAD, 64), _f32),
                       pltpu.SemaphoreType.DMA, pltpu.SemaphoreType.DMA,
                       pltpu.SemaphoreType.DMA, pltpu.SemaphoreType.DMA,
                       pltpu.SemaphoreType.DMA, pltpu.SemaphoreType.DMA],
    )


def kernel(x, edge_index, batch, params):
    src32 = edge_index[0].reshape(32, NCHUNK // 32, CH)
    dst32 = edge_index[1].reshape(32, NCHUNK // 32, CH)
    src16 = edge_index[0].reshape(16, NCHUNK // 16, CH)
    dst16 = edge_index[1].reshape(16, NCHUNK // 16, CH)
    x_pad = jnp.pad(x, ((0, NPAD - N), (0, 0)))
    batch3 = jnp.pad(batch, (0, NPAD - N)).astype(jnp.int32).reshape(
        NGRID, NB, 1)
    zeros64 = jnp.zeros((NPAD, 64), _f32)
    zeros128 = jnp.zeros((NPAD, 128), _f32)

    sc_causal = _make_sc_causal()
    sc_ef = _make_sc_ef()
    sc_cls = _make_sc_cls()

    cl = params["causal"]
    hW = tc_matmul(x_pad, cl[0]["W1"])
    Z = None
    for li in range(3):
        p0, p1 = sc_causal(hW, src32, dst32, zeros64)
        if li < 2:
            _, hW = tc_gin_layer(hW, p0, p1, cl[li], cl[li + 1]["W1"])
        else:
            Z = tc_gin_layer(hW, p0, p1, cl[li], None)

    nm4, fm4, T0, T1, T2, T3 = tc_masks(x_pad, Z, params)
    ef_out = sc_ef(Z, edge_index.reshape(2, 16, NCHUNK // 16, CH))
    em4 = tc_em(ef_out[0].reshape(E, 64), ef_out[1].reshape(E, 64),
                params)
    ew3 = em4.reshape(NCHUNK, CH // 4, 16)

    clf = params["classifier"]
    Ts = [T0, T1, T2, T3]
    mZ = None
    for li in range(3):
        Uall = sc_cls(jnp.stack(Ts), src16, dst16, ew3, zeros64)
        Us = [Uall[0], Uall[1], Uall[2], Uall[3]]
        if li < 2:
            Ts = tc_cls_layer(Ts, Us, clf[li], clf[li + 1]["W1"])
        else:
            mZ = tc_cls_layer(Ts, Us, clf[li], None)

    h_orig, hs_flat, lg_flat = tc_pool(Z, mZ, batch3, params)

    return (lg_flat.reshape(G, 4, 10),
            hs_flat.reshape(G, 4, 64),
            h_orig,
            nm4[:N].reshape(N, 4, 1),
            em4.reshape(E, 4, 1),
            fm4[:N].reshape(N, 4, 128))


# speedup vs baseline: 1.6079x; 1.6079x over previous
import functools

import jax
import jax.numpy as jnp
from jax import lax
from jax.experimental import pallas as pl
from jax.experimental.pallas import tpu as pltpu
from jax.experimental.pallas import tpu_sc as plsc

N = 10000
NPAD = 10240
E = 320000
G = 128
TEMP = 5.0
CH = 100
NCHUNK = E // CH
NB = 512
NGRID = NPAD // NB
EB = 2000
ZR = NPAD // 16

_f32 = jnp.float32
_SDS = jax.ShapeDtypeStruct


def _mesh():
    return plsc.VectorSubcoreMesh(core_axis_name="c", subcore_axis_name="s")


def _mm_body(x_ref, w_ref, o_ref):
    o_ref[...] = jnp.dot(x_ref[...], w_ref[...], preferred_element_type=_f32)


def tc_matmul(x, w):
    n, din = x.shape
    dout = w.shape[1]
    return pl.pallas_call(
        _mm_body,
        grid=(n // NB,),
        in_specs=[pl.BlockSpec((NB, din), lambda i: (i, 0)),
                  pl.BlockSpec((din, dout), lambda i: (0, 0))],
        out_specs=pl.BlockSpec((NB, dout), lambda i: (i, 0)),
        out_shape=_SDS((n, dout), _f32),
    )(x, w)


def _gin_layer_body(last, hw_ref, p0_ref, p1_ref, w2_ref, b1_ref, b2_ref,
                    eps_ref, w1n_ref, *out_refs):
    u = hw_ref[...] * eps_ref[0, 0] + p0_ref[...] + p1_ref[...] + b1_ref[...]
    v = jnp.maximum(u, 0.0)
    h = jnp.maximum(jnp.dot(v, w2_ref[...], preferred_element_type=_f32)
                    + b2_ref[...], 0.0)
    if last:
        out_refs[0][...] = h
    else:
        out_refs[0][...] = h
        out_refs[1][...] = jnp.dot(h, w1n_ref[...], preferred_element_type=_f32)


def tc_gin_layer(hW, p0, p1, p, w1_next):
    last = w1_next is None
    wspec = lambda shp: pl.BlockSpec(shp, lambda i: (0, 0))
    in_specs = [pl.BlockSpec((NB, 64), lambda i: (i, 0))] * 3 + [
        wspec((64, 64)), wspec((1, 64)), wspec((1, 64)),
        pl.BlockSpec(memory_space=pltpu.SMEM), wspec((64, 64))]
    out_specs = [pl.BlockSpec((NB, 64), lambda i: (i, 0))] * (1 if last else 2)
    out_shape = [_SDS((NPAD, 64), _f32)] * (1 if last else 2)
    epsp = (1.0 + p["eps"]).reshape(1, 1).astype(_f32)
    w1n = jnp.zeros((64, 64), _f32) if last else w1_next
    outs = pl.pallas_call(
        functools.partial(_gin_layer_body, last),
        grid=(NGRID,),
        in_specs=in_specs,
        out_specs=out_specs,
        out_shape=out_shape,
    )(hW, p0, p1, p["W2"], p["b1"].reshape(1, 64), p["b2"].reshape(1, 64),
      epsp, w1n)
    return outs[0] if last else outs


def _masks_body(x_ref, z_ref, nw1_ref, nb1_ref, nw2_ref, nb2_ref,
                fw1_ref, fb1_ref, fw2_0, fw2_1, fw2_2, fw2_3,
                fb2_0, fb2_1, fb2_2, fb2_3, w1c_ref,
                nm_ref, fm_ref, t0_ref, t1_ref, t2_ref, t3_ref):
    z = z_ref[...]
    x = x_ref[...]
    h1 = jnp.maximum(jnp.dot(z, nw1_ref[...], preferred_element_type=_f32)
                     + nb1_ref[...], 0.0)
    t = h1 * nw2_ref[...]
    nm_parts = [jnp.sum(t[:, k * 64:(k + 1) * 64], axis=1, keepdims=True)
                for k in range(4)]
    nm = jax.nn.sigmoid((jnp.concatenate(nm_parts, axis=1) + nb2_ref[...])
                        / TEMP)
    nm_ref[...] = nm
    h2 = jnp.maximum(jnp.dot(z, fw1_ref[...], preferred_element_type=_f32)
                     + fb1_ref[...], 0.0)
    fw2 = (fw2_0, fw2_1, fw2_2, fw2_3)
    fb2 = (fb2_0, fb2_1, fb2_2, fb2_3)
    fms = []
    m1s = []
    for k in range(4):
        fmk = jax.nn.sigmoid(
            (jnp.dot(h2[:, k * 64:(k + 1) * 64], fw2[k][...],
                     preferred_element_type=_f32) + fb2[k][...]) / TEMP)
        fms.append(fmk)
        mx = x * nm[:, k:k + 1] * fmk
        m1s.append(jnp.dot(mx, w1c_ref[...], preferred_element_type=_f32))
    fm_ref[...] = jnp.concatenate(fms, axis=1)
    t0_ref[...] = m1s[0]
    t1_ref[...] = m1s[1]
    t2_ref[...] = m1s[2]
    t3_ref[...] = m1s[3]


def tc_masks(x_pad, Z, params):
    nm_p = params["node_masks"]
    fm_p = params["feat_masks"]
    w1c = params["classifier"][0]["W1"]
    nw1 = jnp.concatenate([p["W1"] for p in nm_p], axis=1)
    nb1 = jnp.concatenate([p["b1"] for p in nm_p]).reshape(1, 256)
    nw2 = jnp.concatenate([p["W2"][:, 0] for p in nm_p]).reshape(1, 256)
    nb2 = jnp.stack([p["b2"][0] for p in nm_p]).reshape(1, 4)
    fw1 = jnp.concatenate([p["W1"] for p in fm_p], axis=1)
    fb1 = jnp.concatenate([p["b1"] for p in fm_p]).reshape(1, 256)
    wspec = lambda shp: pl.BlockSpec(shp, lambda i: (0, 0))
    in_specs = [pl.BlockSpec((NB, 128), lambda i: (i, 0)),
                pl.BlockSpec((NB, 64), lambda i: (i, 0)),
                wspec((64, 256)), wspec((1, 256)), wspec((1, 256)),
                wspec((1, 4)), wspec((64, 256)), wspec((1, 256)),
                wspec((64, 128)), wspec((64, 128)), wspec((64, 128)),
                wspec((64, 128)), wspec((1, 128)), wspec((1, 128)),
                wspec((1, 128)), wspec((1, 128)), wspec((128, 64))]
    out_specs = [pl.BlockSpec((NB, 4), lambda i: (i, 0)),
                 pl.BlockSpec((NB, 512), lambda i: (i, 0))] + [
                 pl.BlockSpec((NB, 64), lambda i: (i, 0))] * 4
    out_shape = [_SDS((NPAD, 4), _f32), _SDS((NPAD, 512), _f32)] + [
                 _SDS((NPAD, 64), _f32)] * 4
    args = [x_pad, Z, nw1, nb1, nw2, nb2, fw1, fb1]
    args += [fm_p[k]["W2"] for k in range(4)]
    args += [fm_p[k]["b2"].reshape(1, 128) for k in range(4)]
    args += [w1c]
    return pl.pallas_call(
        _masks_body, grid=(NGRID,), in_specs=in_specs, out_specs=out_specs,
        out_shape=out_shape)(*args)


def _em_body(efs_ref, efd_ref, wa_ref, wb_ref, b1_ref, w2_ref, b2_ref,
             em_ref):
    h = jnp.maximum(
        jnp.dot(efs_ref[...], wa_ref[...], preferred_element_type=_f32)
        + jnp.dot(efd_ref[...], wb_ref[...], preferred_element_type=_f32)
        + b1_ref[...], 0.0)
    t = h * w2_ref[...]
    parts = [jnp.sum(t[:, k * 64:(k + 1) * 64], axis=1, keepdims=True)
             for k in range(4)]
    em_ref[...] = jax.nn.sigmoid(
        (jnp.concatenate(parts, axis=1) + b2_ref[...]) / TEMP)


def tc_em(efs, efd, params):
    em_p = params["edge_masks"]
    wa = jnp.concatenate([p["W1"][:64] for p in em_p], axis=1)
    wb = jnp.concatenate([p["W1"][64:] for p in em_p], axis=1)
    b1 = jnp.concatenate([p["b1"] for p in em_p]).reshape(1, 256)
    w2 = jnp.concatenate([p["W2"][:, 0] for p in em_p]).reshape(1, 256)
    b2 = jnp.stack([p["b2"][0] for p in em_p]).reshape(1, 4)
    wspec = lambda shp: pl.BlockSpec(shp, lambda i: (0, 0))
    return pl.pallas_call(
        _em_body,
        grid=(E // EB,),
        in_specs=[pl.BlockSpec((EB, 64), lambda i: (i, 0)),
                  pl.BlockSpec((EB, 64), lambda i: (i, 0)),
                  wspec((64, 256)), wspec((64, 256)), wspec((1, 256)),
                  wspec((1, 256)), wspec((1, 4))],
        out_specs=pl.BlockSpec((EB, 4), lambda i: (i, 0)),
        out_shape=_SDS((E, 4), _f32),
    )(efs, efd, wa, wb, b1, w2, b2)


def _cls_layer_body(last, t0_ref, t1_ref, t2_ref, t3_ref,
                    u0_ref, u1_ref, u2_ref, u3_ref, w2_ref, b1_ref,
                    b2_ref, eps_ref, w1n_ref, *out_refs):
    w2 = w2_ref[...]
    b2 = b2_ref[...]
    t_refs = (t0_ref, t1_ref, t2_ref, t3_ref)
    u_refs = (u0_ref, u1_ref, u2_ref, u3_ref)
    hs = []
    for k in range(4):
        u = t_refs[k][...] * eps_ref[0, 0] + u_refs[k][...] + b1_ref[...]
        v = jnp.maximum(u, 0.0)
        hs.append(jnp.maximum(jnp.dot(v, w2, preferred_element_type=_f32)
                              + b2, 0.0))
    if last:
        out_refs[0][...] = jnp.concatenate(hs, axis=1)
    else:
        w1n = w1n_ref[...]
        for k in range(4):
            out_refs[k][...] = jnp.dot(hs[k], w1n,
                                       preferred_element_type=_f32)


def tc_cls_layer(Ts, Us, p, w1_next):
    last = w1_next is None
    wspec = lambda shp: pl.BlockSpec(shp, lambda i: (0, 0))
    in_specs = [pl.BlockSpec((NB, 64), lambda i: (i, 0))] * 8 + [
        wspec((64, 64)), wspec((1, 64)), wspec((1, 64)),
        pl.BlockSpec(memory_space=pltpu.SMEM), wspec((64, 64))]
    if last:
        out_specs = [pl.BlockSpec((NB, 256), lambda i: (i, 0))]
        out_shape = [_SDS((NPAD, 256), _f32)]
    else:
        out_specs = [pl.BlockSpec((NB, 64), lambda i: (i, 0))] * 4
        out_shape = [_SDS((NPAD, 64), _f32)] * 4
    epsp = (1.0 + p["eps"]).reshape(1, 1).astype(_f32)
    w1n = jnp.zeros((64, 64), _f32) if last else w1_next
    outs = pl.pallas_call(
        functools.partial(_cls_layer_body, last),
        grid=(NGRID,),
        in_specs=in_specs,
        out_specs=out_specs,
        out_shape=out_shape,
    )(*Ts, *Us, p["W2"], p["b1"].reshape(1, 64), p["b2"].reshape(1, 64),
      epsp, w1n)
    return outs[0] if last else outs


def _pool_body(z_ref, mz_ref, b_ref, wl_ref, bl_ref,
               ho_ref, hs_ref, lg_ref, sz_acc, sm_acc, cnt_acc):
    i = pl.program_id(0)

    @pl.when(i == 0)
    def _init():
        sz_acc[...] = jnp.zeros_like(sz_acc)
        sm_acc[...] = jnp.zeros_like(sm_acc)
        cnt_acc[...] = jnp.zeros_like(cnt_acc)

    bcol = jnp.reshape(b_ref[...], (NB, 1))
    iota = lax.broadcasted_iota(jnp.int32, (NB, G), 1)
    rows = i * NB + lax.broadcasted_iota(jnp.int32, (NB, 1), 0)
    valid = (rows < N).astype(_f32)
    oh = (bcol == iota).astype(_f32) * valid
    dn = (((0,), (0,)), ((), ()))
    sz_acc[...] += lax.dot_general(oh, z_ref[...], dn,
                                   preferred_element_type=_f32)
    sm_acc[...] += lax.dot_general(oh, mz_ref[...], dn,
                                   preferred_element_type=_f32)
    cnt_acc[...] += lax.dot_general(oh, valid, dn,
                                    preferred_element_type=_f32)

    @pl.when(i == NGRID - 1)
    def _fin():
        cnt = jnp.maximum(cnt_acc[...], 1.0)
        ho = sz_acc[...] / cnt
        hs = sm_acc[...] / cnt
        ho_ref[...] = ho
        hs_ref[...] = hs
        wl = wl_ref[...]
        bl = bl_ref[...]
        lgs = [jnp.dot(hs[:, k * 64:(k + 1) * 64], wl[:, k * 10:(k + 1) * 10],
                       preferred_element_type=_f32) + bl[:, k * 10:(k + 1) * 10]
               for k in range(4)]
        lg_ref[...] = jnp.concatenate(lgs, axis=1)


def tc_pool(Z, mZ, batch3, params):
    wl = jnp.concatenate([params["classifiers"][k]["W"] for k in range(4)],
                         axis=1)
    bl = jnp.concatenate([params["classifiers"][k]["b"] for k in range(4)]
                         ).reshape(1, 40)
    wspec = lambda shp: pl.BlockSpec(shp, lambda i: (0, 0))
    return pl.pallas_call(
        _pool_body,
        grid=(NGRID,),
        in_specs=[pl.BlockSpec((NB, 64), lambda i: (i, 0)),
                  pl.BlockSpec((NB, 256), lambda i: (i, 0)),
                  pl.BlockSpec((1, NB, 1), lambda i: (i, 0, 0)),
                  wspec((64, 40)), wspec((1, 40))],
        out_specs=[pl.BlockSpec((G, 64), lambda i: (0, 0)),
                   pl.BlockSpec((G, 256), lambda i: (0, 0)),
                   pl.BlockSpec((G, 40), lambda i: (0, 0))],
        out_shape=[_SDS((G, 64), _f32), _SDS((G, 256), _f32),
                   _SDS((G, 40), _f32)],
        scratch_shapes=[pltpu.VMEM((G, 64), _f32), pltpu.VMEM((G, 256), _f32),
                        pltpu.VMEM((G, 1), _f32)],
    )(Z, mZ, batch3, wl, bl)


_GDN = lax.GatherDimensionNumbers(offset_dims=(), collapsed_slice_dims=(0,),
                                  start_index_map=(0,))


def _lane_bcast(w, idx):
    return lax.gather(w, idx[:, None], _GDN, (1,),
                      mode=lax.GatherScatterMode.PROMISE_IN_BOUNDS)

S = 2


def _make_sc_causal():
    cpw = NCHUNK // 32
    nsup = cpw // S

    def body(tbl, srcr, dstr, zr, out0, out1,
             src_v, dst_v, rows0, rows1, accum, sem0, sem1, ssem0, ssem1):
        c = lax.axis_index("c")
        s = lax.axis_index("s")
        wid = s * 2 + c
        pltpu.sync_copy(zr.at[pl.ds(s * ZR, ZR)], accum.at[pl.ds(s * ZR, ZR)])
        pltpu.sync_copy(srcr.at[wid], src_v)
        pltpu.sync_copy(dstr.at[wid], dst_v)
        plsc.subcore_barrier()

        def g_start(J, buf, sem):
            for k in range(S):
                pltpu.async_copy(tbl.at[src_v.at[J * S + k]], buf.at[k], sem)

        def g_wait(buf, sem):
            for k in range(S):
                pltpu.make_async_copy(tbl.at[src_v.at[0]], buf.at[k],
                                      sem).wait()

        def s_start(J, buf, ssem):
            for k in range(S):
                pltpu.async_copy(buf.at[k], accum.at[dst_v.at[J * S + k]],
                                 ssem, add=True)

        def s_wait(buf, ssem):
            for k in range(S):
                pltpu.make_async_copy(buf.at[k], accum.at[dst_v.at[0]],
                                      ssem).wait()

        g_start(0, rows0, sem0)
        g_start(1, rows1, sem1)

        def pair(i, carry):
            J0 = i * 2
            g_wait(rows0, sem0)
            s_start(J0, rows0, ssem0)
            s_wait(rows0, ssem0)

            @pl.when(J0 + 2 < nsup)
            def _():
                g_start(J0 + 2, rows0, sem0)

            g_wait(rows1, sem1)
            s_start(J0 + 1, rows1, ssem1)
            s_wait(rows1, ssem1)

            @pl.when(J0 + 3 < nsup)
            def _():
                g_start(J0 + 3, rows1, sem1)
            return carry

        lax.fori_loop(0, nsup // 2, pair, 0)
        plsc.subcore_barrier()
        sl = pl.ds(s * ZR, ZR)

        @pl.when(c == 0)
        def _():
            pltpu.sync_copy(accum.at[sl], out0.at[sl])

        @pl.when(c == 1)
        def _():
            pltpu.sync_copy(accum.at[sl], out1.at[sl])

    return pl.kernel(
        body,
        out_type=(_SDS((NPAD, 64), _f32), _SDS((NPAD, 64), _f32)),
        mesh=_mesh(),
        compiler_params=pltpu.CompilerParams(use_tc_tiling_on_sc=False),
        scratch_types=[pltpu.VMEM((NCHUNK // 32, CH), jnp.int32),
                       pltpu.VMEM((NCHUNK // 32, CH), jnp.int32),
                       pltpu.VMEM((S, CH, 64), _f32),
                       pltpu.VMEM((S, CH, 64), _f32),
                       pltpu.VMEM_SHARED((NPAD, 64), _f32),
                       pltpu.SemaphoreType.DMA, pltpu.SemaphoreType.DMA,
                       pltpu.SemaphoreType.DMA, pltpu.SemaphoreType.DMA],
    )


def _make_sc_ef():
    EFCH = 80
    cpt = (E // EFCH) // 16
    nsup = cpt // 2

    def body(tbl, idxr, out, idx_v, b0, b1, sem0, sem1, wsem0, wsem1):
        c = lax.axis_index("c")
        s = lax.axis_index("s")
        pltpu.sync_copy(idxr.at[c, s], idx_v)
        outc = out.at[c]
        ebase = s * cpt * EFCH

        def g_start(J, buf, sem):
            for k in range(2):
                pltpu.async_copy(tbl.at[idx_v.at[J * 2 + k]], buf.at[k], sem)

        def g_wait(buf, sem):
            for k in range(2):
                pltpu.make_async_copy(tbl.at[idx_v.at[0]], buf.at[k],
                                      sem).wait()

        def w_start(J, buf, wsem):
            for k in range(2):
                pltpu.async_copy(buf.at[k],
                                 outc.at[pl.ds(ebase + (J * 2 + k) * EFCH,
                                               EFCH)], wsem)

        def w_wait(buf, wsem):
            for k in range(2):
                pltpu.make_async_copy(buf.at[k],
                                      outc.at[pl.ds(0, EFCH)], wsem).wait()

        g_start(0, b0, sem0)
        g_start(1, b1, sem1)

        def pair(i, carry):
            J0 = i * 2
            g_wait(b0, sem0)
            w_start(J0, b0, wsem0)
            w_wait(b0, wsem0)

            @pl.when(J0 + 2 < nsup)
            def _():
                g_start(J0 + 2, b0, sem0)

            g_wait(b1, sem1)
            w_start(J0 + 1, b1, wsem1)
            w_wait(b1, wsem1)

            @pl.when(J0 + 3 < nsup)
            def _():
                g_start(J0 + 3, b1, sem1)
            return carry

        lax.fori_loop(0, nsup // 2, pair, 0)
        g_wait(b0, sem0)
        w_start(nsup - 1, b0, wsem0)
        w_wait(b0, wsem0)

    return pl.kernel(
        body,
        out_type=_SDS((2, E, 64), _f32),
        mesh=_mesh(),
        compiler_params=pltpu.CompilerParams(use_tc_tiling_on_sc=False),
        scratch_types=[pltpu.VMEM(((E // EFCH) // 16, EFCH), jnp.int32),
                       pltpu.VMEM((2, EFCH, 64), _f32),
                       pltpu.VMEM((2, EFCH, 64), _f32),
                       pltpu.SemaphoreType.DMA, pltpu.SemaphoreType.DMA,
                       pltpu.SemaphoreType.DMA, pltpu.SemaphoreType.DMA],
    )


def _make_sc_cls():
    cpt = NCHUNK // 16
    wrows = CH // 4
    nsup = cpt // S

    def body(tab, srcr, dstr, ewr, zr, uout,
             src_v, dst_v, rows0, rows1, ew0, ew1, accum,
             sem0, sem1, semw0, semw1, ssem0, ssem1):
        c = lax.axis_index("c")
        s = lax.axis_index("s")
        pltpu.sync_copy(srcr.at[s], src_v)
        pltpu.sync_copy(dstr.at[s], dst_v)

        def s_start(J, buf, ssem):
            for k in range(S):
                pltpu.async_copy(buf.at[k], accum.at[dst_v.at[J * S + k]],
                                 ssem, add=True)

        def s_wait(buf, ssem):
            for k in range(S):
                pltpu.make_async_copy(buf.at[k], accum.at[dst_v.at[0]],
                                      ssem).wait()

        def scale_k(buf, ewb, kidx):
            for kk in range(S):
                def grp(g, carry):
                    w = ewb[kk, g]
                    for i in range(4):
                        wk = _lane_bcast(w, jnp.full((16,), 4 * i + kidx,
                                                     jnp.int32))
                        r = g * 4 + i
                        for q in range(4):
                            sl = pl.ds(q * 16, 16)
                            buf[kk, r, sl] = buf[kk, r, sl] * wk
                    return carry

                lax.fori_loop(0, wrows, grp, 0)

        for p in range(2):
            tc_ = tab.at[2 * p + c]

            def g_start(J, buf, ewb, sem, semw, tc_=tc_):
                for k in range(S):
                    pltpu.async_copy(tc_.at[src_v.at[J * S + k]], buf.at[k],
                                     sem)
                pltpu.async_copy(ewr.at[pl.ds(s * cpt + J * S, S)], ewb, semw)

            def g_wait(buf, ewb, sem, semw, tc_=tc_):
                for k in range(S):
                    pltpu.make_async_copy(tc_.at[src_v.at[0]], buf.at[k],
                                          sem).wait()
                pltpu.make_async_copy(ewr.at[pl.ds(0, S)], ewb, semw).wait()

            def scale(buf, ewb, p=p):
                @pl.when(c == 0)
                def _():
                    scale_k(buf, ewb, 2 * p)

                @pl.when(c == 1)
                def _():
                    scale_k(buf, ewb, 2 * p + 1)

            plsc.subcore_barrier()
            pltpu.sync_copy(zr.at[pl.ds(s * ZR, ZR)],
                            accum.at[pl.ds(s * ZR, ZR)])
            plsc.subcore_barrier()

            g_start(0, rows0, ew0, sem0, semw0)
            g_start(1, rows1, ew1, sem1, semw1)

            def pair(i, carry, g_start=g_start, g_wait=g_wait, scale=scale):
                J0 = i * 2
                g_wait(rows0, ew0, sem0, semw0)
                scale(rows0, ew0)
                s_start(J0, rows0, ssem0)
                s_wait(rows0, ssem0)

                @pl.when(J0 + 2 < nsup)
                def _():
                    g_start(J0 + 2, rows0, ew0, sem0, semw0)

                g_wait(rows1, ew1, sem1, semw1)
                scale(rows1, ew1)
                s_start(J0 + 1, rows1, ssem1)
                s_wait(rows1, ssem1)

                @pl.when(J0 + 3 < nsup)
                def _():
                    g_start(J0 + 3, rows1, ew1, sem1, semw1)
                return carry

            lax.fori_loop(0, nsup // 2, pair, 0)
            plsc.subcore_barrier()
            sl = pl.ds(s * ZR, ZR)
            pltpu.sync_copy(accum.at[sl], uout.at[2 * p + c, sl])

    return pl.kernel(
        body,
        out_type=_SDS((4, NPAD, 64), _f32),
        mesh=_mesh(),
        compiler_params=pltpu.CompilerParams(use_tc_tiling_on_sc=False),
        scratch_types=[pltpu.VMEM((NCHUNK // 16, CH), jnp.int32),
                       pltpu.VMEM((NCHUNK // 16, CH), jnp.int32),
                       pltpu.VMEM((S, CH, 64), _f32),
                       pltpu.VMEM((S, CH, 64), _f32),
                       pltpu.VMEM((S, CH // 4, 16), _f32),
                       pltpu.VMEM((S, CH // 4, 16), _f32),
                       pltpu.VMEM_SHARED((NPAD, 64), _f32),
                       pltpu.SemaphoreType.DMA, pltpu.SemaphoreType.DMA,
                       pltpu.SemaphoreType.DMA, pltpu.SemaphoreType.DMA,
                       pltpu.SemaphoreType.DMA, pltpu.SemaphoreType.DMA],
    )


def kernel(x, edge_index, batch, params):
    src32 = edge_index[0].reshape(32, NCHUNK // 32, CH)
    dst32 = edge_index[1].reshape(32, NCHUNK // 32, CH)
    src16 = edge_index[0].reshape(16, NCHUNK // 16, CH)
    dst16 = edge_index[1].reshape(16, NCHUNK // 16, CH)
    x_pad = jnp.pad(x, ((0, NPAD - N), (0, 0)))
    batch3 = jnp.pad(batch, (0, NPAD - N)).astype(jnp.int32).reshape(
        NGRID, NB, 1)
    zeros64 = jnp.zeros((NPAD, 64), _f32)
    zeros128 = jnp.zeros((NPAD, 128), _f32)

    sc_causal = _make_sc_causal()
    sc_ef = _make_sc_ef()
    sc_cls = _make_sc_cls()

    cl = params["causal"]
    hW = tc_matmul(x_pad, cl[0]["W1"])
    Z = None
    for li in range(3):
        p0, p1 = sc_causal(hW, src32, dst32, zeros64)
        if li < 2:
            _, hW = tc_gin_layer(hW, p0, p1, cl[li], cl[li + 1]["W1"])
        else:
            Z = tc_gin_layer(hW, p0, p1, cl[li], None)

    nm4, fm4, T0, T1, T2, T3 = tc_masks(x_pad, Z, params)
    ef_out = sc_ef(Z, edge_index.reshape(2, 16, (E // 80) // 16, 80))
    em4 = tc_em(ef_out[0], ef_out[1], params)
    ew3 = em4.reshape(NCHUNK, CH // 4, 16)

    clf = params["classifier"]
    Ts = [T0, T1, T2, T3]
    mZ = None
    for li in range(3):
        Uall = sc_cls(jnp.stack(Ts), src16, dst16, ew3, zeros64)
        Us = [Uall[0], Uall[1], Uall[2], Uall[3]]
        if li < 2:
            Ts = tc_cls_layer(Ts, Us, clf[li], clf[li + 1]["W1"])
        else:
            mZ = tc_cls_layer(Ts, Us, clf[li], None)

    h_orig, hs_flat, lg_flat = tc_pool(Z, mZ, batch3, params)

    return (lg_flat.reshape(G, 4, 10),
            hs_flat.reshape(G, 4, 64),
            h_orig,
            nm4[:N].reshape(N, 4, 1),
            em4.reshape(E, 4, 1),
            fm4[:N].reshape(N, 4, 128))
